# baseline (speedup 1.0000x reference)
import jax
import jax.numpy as jnp
from jax.experimental import pallas as pl
from jax.experimental.pallas import tpu as pltpu


_TOPK = 2
_TPAD = 8
_FB = 1024
_SB = 256


def _expert_body(x8_ref, w1_ref, b1_ref, w2_ref, b2_ref, y_ref):
    f = pl.program_id(1)
    h = jnp.dot(x8_ref[...], w1_ref[0], preferred_element_type=jnp.float32)
    h = h + b1_ref[0]
    h = 0.5 * h * (1.0 + jax.lax.erf(h * jnp.float32(0.7071067811865476)))
    contrib = jnp.dot(h, w2_ref[0], preferred_element_type=jnp.float32)

    @pl.when(f == 0)
    def _init():
        y_ref[0] = contrib + b2_ref[0]

    @pl.when(f != 0)
    def _acc():
        y_ref[0] = y_ref[0] + contrib


def _routing_body(x_ref, wg_ref, bg_ref, y_ref, o_ref):
    logits = jnp.dot(x_ref[...], wg_ref[...], preferred_element_type=jnp.float32)
    logits = logits + bg_ref[...]
    m = jnp.max(logits, axis=-1, keepdims=True)
    p = jnp.exp(logits - m)
    p = p / jnp.sum(p, axis=-1, keepdims=True)

    a1 = jnp.argmax(p, axis=-1, keepdims=True)
    v1 = jnp.max(p, axis=-1, keepdims=True)
    e_iota = jax.lax.broadcasted_iota(jnp.int32, p.shape, 1)
    p_masked = jnp.where(e_iota == a1, -jnp.inf, p)
    a2 = jnp.argmax(p_masked, axis=-1, keepdims=True)
    v2 = jnp.max(p_masked, axis=-1, keepdims=True)

    t = jax.lax.broadcasted_iota(jnp.int32, (p.shape[0], y_ref.shape[0]), 1)
    c = jnp.where(t == a1 * _TPAD, v1, 0.0) + jnp.where(t == a2 * _TPAD + 1, v2, 0.0)
    o_ref[...] = jnp.dot(c, y_ref[...], preferred_element_type=jnp.float32)


def kernel(x, W1, b1, W2, b2, Wg, bg):
    B, S, D = x.shape
    E, _, F = W1.shape
    x2d = x.reshape(S, D)
    x8 = x2d[:_TPAD]
    b1r = b1.reshape(E, 1, F)
    b2r = b2.reshape(E, 1, D)
    bgr = bg.reshape(1, E)

    nf = F // _FB
    y = pl.pallas_call(
        _expert_body,
        grid=(E, nf),
        in_specs=[
            pl.BlockSpec((_TPAD, D), lambda e, f: (0, 0)),
            pl.BlockSpec((1, D, _FB), lambda e, f: (e, 0, f)),
            pl.BlockSpec((1, 1, _FB), lambda e, f: (e, 0, f)),
            pl.BlockSpec((1, _FB, D), lambda e, f: (e, f, 0)),
            pl.BlockSpec((1, 1, D), lambda e, f: (e, 0, 0)),
        ],
        out_specs=pl.BlockSpec((1, _TPAD, D), lambda e, f: (e, 0, 0)),
        out_shape=jax.ShapeDtypeStruct((E, _TPAD, D), jnp.float32),
        compiler_params=pltpu.CompilerParams(
            dimension_semantics=("parallel", "arbitrary"),
        ),
    )(x8, W1, b1r, W2, b2r)

    yflat = y.reshape(E * _TPAD, D)

    out = pl.pallas_call(
        _routing_body,
        grid=(S // _SB,),
        in_specs=[
            pl.BlockSpec((_SB, D), lambda s: (s, 0)),
            pl.BlockSpec((D, E), lambda s: (0, 0)),
            pl.BlockSpec((1, E), lambda s: (0, 0)),
            pl.BlockSpec((E * _TPAD, D), lambda s: (0, 0)),
        ],
        out_specs=pl.BlockSpec((_SB, D), lambda s: (s, 0)),
        out_shape=jax.ShapeDtypeStruct((S, D), jnp.float32),
        compiler_params=pltpu.CompilerParams(
            dimension_semantics=("parallel",),
        ),
    )(x2d, Wg, bgr, yflat)

    return out.reshape(B, S, D)


# speedup vs baseline: 1.0431x; 1.0431x over previous
import jax
import jax.numpy as jnp
from jax.experimental import pallas as pl
from jax.experimental.pallas import tpu as pltpu


_TOPK = 2
_TPAD = 8
_FB = 1024


def _fused_body(xs_ref, x8_ref, w1_ref, b1_ref, w2_ref, b2_ref, wg_ref, bg_ref,
                o_ref, c_ref, y_ref):
    e = pl.program_id(0)
    f = pl.program_id(1)
    ne = pl.num_programs(0)
    nf = pl.num_programs(1)
    sb = xs_ref.shape[0]

    @pl.when(f == 0)
    def _gating():
        logits = jnp.dot(xs_ref[...], wg_ref[...],
                         preferred_element_type=jnp.float32) + bg_ref[...]
        m = jnp.max(logits, axis=-1, keepdims=True)
        p = jnp.exp(logits - m)
        p = p / jnp.sum(p, axis=-1, keepdims=True)
        a1 = jnp.argmax(p, axis=-1, keepdims=True)
        v1 = jnp.max(p, axis=-1, keepdims=True)
        e_iota = jax.lax.broadcasted_iota(jnp.int32, p.shape, 1)
        p_masked = jnp.where(e_iota == a1, -jnp.inf, p)
        a2 = jnp.argmax(p_masked, axis=-1, keepdims=True)
        v2 = jnp.max(p_masked, axis=-1, keepdims=True)
        t = jax.lax.broadcasted_iota(jnp.int32, (sb, c_ref.shape[1]), 1)
        c = (jnp.where(t == a1 * _TPAD, v1, 0.0)
             + jnp.where(t == a2 * _TPAD + 1, v2, 0.0))
        c_ref[pl.ds(e * sb, sb), :] = c

    h = jnp.dot(x8_ref[...], w1_ref[0], preferred_element_type=jnp.float32)
    h = h + b1_ref[0]
    h = 0.5 * h * (1.0 + jax.lax.erf(h * jnp.float32(0.7071067811865476)))
    contrib = jnp.dot(h, w2_ref[0], preferred_element_type=jnp.float32)

    @pl.when(f == 0)
    def _init():
        y_ref[pl.ds(e * _TPAD, _TPAD), :] = contrib + b2_ref[0]

    @pl.when(f != 0)
    def _acc():
        y_ref[pl.ds(e * _TPAD, _TPAD), :] = (
            y_ref[pl.ds(e * _TPAD, _TPAD), :] + contrib)

    @pl.when((e == ne - 1) & (f == nf - 1))
    def _combine():
        o_ref[...] = jnp.dot(c_ref[...], y_ref[...],
                             preferred_element_type=jnp.float32)


def kernel(x, W1, b1, W2, b2, Wg, bg):
    B, S, D = x.shape
    E, _, F = W1.shape
    x2d = x.reshape(S, D)
    x8 = x2d[:_TPAD]
    b1r = b1.reshape(E, 1, F)
    b2r = b2.reshape(E, 1, D)
    bgr = bg.reshape(1, E)

    nf = F // _FB
    sb = S // E

    out = pl.pallas_call(
        _fused_body,
        grid=(E, nf),
        in_specs=[
            pl.BlockSpec((sb, D), lambda e, f: (e, 0)),
            pl.BlockSpec((_TPAD, D), lambda e, f: (0, 0)),
            pl.BlockSpec((1, D, _FB), lambda e, f: (e, 0, f)),
            pl.BlockSpec((1, 1, _FB), lambda e, f: (e, 0, f)),
            pl.BlockSpec((1, _FB, D), lambda e, f: (e, f, 0)),
            pl.BlockSpec((1, 1, D), lambda e, f: (e, 0, 0)),
            pl.BlockSpec((D, E), lambda e, f: (0, 0)),
            pl.BlockSpec((1, E), lambda e, f: (0, 0)),
        ],
        out_specs=pl.BlockSpec((S, D), lambda e, f: (0, 0)),
        out_shape=jax.ShapeDtypeStruct((S, D), jnp.float32),
        scratch_shapes=[
            pltpu.VMEM((S, E * _TPAD), jnp.float32),
            pltpu.VMEM((E * _TPAD, D), jnp.float32),
        ],
        compiler_params=pltpu.CompilerParams(
            dimension_semantics=("arbitrary", "arbitrary"),
        ),
    )(x2d, x8, W1, b1r, W2, b2r, Wg, bgr)

    return out.reshape(B, S, D)
